# baseline (speedup 1.0000x reference)
import functools
import jax
import jax.numpy as jnp
from jax import lax
from jax.experimental import pallas as pl
from jax.experimental.pallas import tpu as pltpu
from jax.experimental.pallas import tpu_sc as plsc

_B = 4
_S = 8192
_D = 1024
_C = 16
_E = _C * _D
_L = 16
_UNROLL = 8
_RING = 4
_DIST = 2


def _sc_body(in_hbm, pos_hbm, out_hbm, *scratch):
    accs = scratch[0:_RING]
    poss = scratch[_RING:_RING + 2]
    in_sems = scratch[_RING + 2:_RING + 2 + _RING]
    out_sems = scratch[_RING + 2 + _RING:_RING + 2 + 2 * _RING]
    pos_sems = scratch[_RING + 2 + 2 * _RING:]

    info = plsc.get_sparse_core_info()
    nc = info.num_cores
    wid = lax.axis_index("s") * nc + lax.axis_index("c")
    rows_per_w = _S // (nc * info.num_subcores)
    G = rows_per_w // _C
    wbase = wid * rows_per_w * _D

    def pos_copy(g, par):
        return pltpu.make_async_copy(
            pos_hbm.at[pl.ds(wbase + g * _E, _E)], poss[par], pos_sems[par])

    def in_copy(g, b, slot):
        off = b * _S * _D + wbase + g * _E
        return pltpu.make_async_copy(
            in_hbm.at[pl.ds(off, _E)], accs[slot], in_sems[slot])

    def out_copy(g, b, slot):
        off = b * _S * _D + wbase + g * _E
        return pltpu.make_async_copy(
            accs[slot], out_hbm.at[pl.ds(off, _E)], out_sems[slot])

    pos_copy(0, 0).start()
    for b in range(_DIST):
        in_copy(0, b, b % _RING).start()

    def outer(g2, _):
        for gg in range(2):
            g = g2 * 2 + gg
            par = gg
            for b in range(_B):
                slot = (gg * _B + b) % _RING
                if b == 0:
                    pos_copy(g, par).wait()

                    @pl.when(g + 1 < G)
                    def _():
                        pos_copy(g + 1, 1 - par).start()

                in_copy(g, b, slot).wait()

                op = b + _DIST
                bp, gp_off = op % _B, op // _B
                orl = b + _DIST - _RING
                br, gr_off = orl % _B, (orl - (orl % _B)) // _B

                @pl.when(g + gp_off < G)
                def _():
                    @pl.when(g + gr_off >= 0)
                    def _():
                        out_copy(g + gr_off, br, (slot + _DIST) % _RING).wait()

                    in_copy(g + gp_off, bp, (slot + _DIST) % _RING).start()

                acc = accs[slot]
                pos = poss[par]

                def add_block(i, _):
                    base = i * (_L * _UNROLL)
                    for k in range(_UNROLL):
                        o = base + k * _L
                        acc[pl.ds(o, _L)] = acc[pl.ds(o, _L)] + pos[pl.ds(o, _L)]
                    return ()

                lax.fori_loop(0, _E // (_L * _UNROLL), add_block, (), unroll=False)
                out_copy(g, b, slot).start()
        return ()

    lax.fori_loop(0, G // 2, outer, (), unroll=False)

    for k in range(_RING):
        t = G * _B - _RING + k
        g, b = t // _B, t % _B
        slot = ((g % 2) * _B + b) % _RING
        out_copy(g, b, slot).wait()


def kernel(inputs, pos_table):
    inputs = inputs.astype(jnp.float32)
    B, S, D = inputs.shape
    flat = inputs.reshape(B * S * D)
    posf = pos_table.reshape(S * D)

    mesh = plsc.VectorSubcoreMesh(core_axis_name="c", subcore_axis_name="s")
    scratch = (
        [pltpu.VMEM((_E,), jnp.float32) for _ in range(_RING)]
        + [pltpu.VMEM((_E,), jnp.float32) for _ in range(2)]
        + [pltpu.SemaphoreType.DMA for _ in range(2 * _RING + 2)]
    )
    sc_add = functools.partial(
        pl.kernel,
        mesh=mesh,
        out_type=jax.ShapeDtypeStruct((B * S * D,), jnp.float32),
        scratch_types=scratch,
    )(_sc_body)
    out = sc_add(flat, posf)
    return out.reshape(B, S, D)


# speedup vs baseline: 1.0457x; 1.0457x over previous
import functools
import jax
import jax.numpy as jnp
from jax import lax
from jax.experimental import pallas as pl
from jax.experimental.pallas import tpu as pltpu
from jax.experimental.pallas import tpu_sc as plsc

_B = 4
_S = 8192
_D = 1024

_BS = 2048


def _tc_body(in_ref, pos_ref, out_ref):
    out_ref[...] = in_ref[...] + pos_ref[...]


def _tc_part(flat2d, pos_table, n_batches):
    n_s = _S // _BS
    return pl.pallas_call(
        _tc_body,
        grid=(n_s, n_batches),
        in_specs=[
            pl.BlockSpec((_BS, _D), lambda s, b: (b * n_s + s, 0)),
            pl.BlockSpec((_BS, _D), lambda s, b: (s, 0)),
        ],
        out_specs=pl.BlockSpec((_BS, _D), lambda s, b: (b * n_s + s, 0)),
        out_shape=jax.ShapeDtypeStruct((n_batches * _S, _D), jnp.float32),
        compiler_params=pltpu.CompilerParams(
            dimension_semantics=("arbitrary", "arbitrary"),
        ),
    )(flat2d, pos_table)


_C = 8
_E = _C * _D
_L = 16
_UNROLL = 8
_RING = 4
_DIST = 2


def _sc_body(in_hbm, pos_hbm, out_hbm, *scratch):
    accs = scratch[0:_RING]
    poss = scratch[_RING:2 * _RING]
    in_sems = scratch[2 * _RING:3 * _RING]
    pos_sems = scratch[3 * _RING:4 * _RING]
    out_sems = scratch[4 * _RING:5 * _RING]

    info = plsc.get_sparse_core_info()
    nc = info.num_cores
    wid = lax.axis_index("s") * nc + lax.axis_index("c")
    rows_per_w = _S // (nc * info.num_subcores)
    G = rows_per_w // _C
    wbase = wid * rows_per_w * _D
    ibase = (_B - 1) * _S * _D + wbase

    def in_copy(g, slot):
        return pltpu.make_async_copy(
            in_hbm.at[pl.ds(ibase + g * _E, _E)], accs[slot], in_sems[slot])

    def pos_copy(g, slot):
        return pltpu.make_async_copy(
            pos_hbm.at[pl.ds(wbase + g * _E, _E)], poss[slot], pos_sems[slot])

    def out_copy(g, slot):
        return pltpu.make_async_copy(
            accs[slot], out_hbm.at[pl.ds(wbase + g * _E, _E)], out_sems[slot])

    for g in range(_DIST):
        in_copy(g, g).start()
        pos_copy(g, g).start()

    def outer(g4, _):
        for gs in range(_RING):
            g = g4 * _RING + gs
            slot = gs
            in_copy(g, slot).wait()
            pos_copy(g, slot).wait()

            nslot = (slot + _DIST) % _RING

            @pl.when(g + _DIST < G)
            def _():
                @pl.when(g + _DIST - _RING >= 0)
                def _():
                    out_copy(g + _DIST - _RING, nslot).wait()

                in_copy(g + _DIST, nslot).start()
                pos_copy(g + _DIST, nslot).start()

            acc = accs[slot]
            pos = poss[slot]

            def add_block(i, _):
                base = i * (_L * _UNROLL)
                for k in range(_UNROLL):
                    o = base + k * _L
                    acc[pl.ds(o, _L)] = acc[pl.ds(o, _L)] + pos[pl.ds(o, _L)]
                return ()

            lax.fori_loop(0, _E // (_L * _UNROLL), add_block, (), unroll=False)
            out_copy(g, slot).start()
        return ()

    lax.fori_loop(0, G // _RING, outer, (), unroll=False)

    for k in range(_RING):
        g = G - _RING + k
        out_copy(g, g % _RING).wait()


def _sc_part(in_flat, pos_flat):
    mesh = plsc.VectorSubcoreMesh(core_axis_name="c", subcore_axis_name="s")
    scratch = (
        [pltpu.VMEM((_E,), jnp.float32) for _ in range(2 * _RING)]
        + [pltpu.SemaphoreType.DMA for _ in range(3 * _RING)]
    )
    sc_add = functools.partial(
        pl.kernel,
        mesh=mesh,
        out_type=jax.ShapeDtypeStruct((_S * _D,), jnp.float32),
        scratch_types=scratch,
    )(_sc_body)
    return sc_add(in_flat, pos_flat)


def kernel(inputs, pos_table):
    inputs = inputs.astype(jnp.float32)
    B, S, D = inputs.shape
    flat = inputs.reshape(B * S, D)

    tc_out = _tc_part(flat, pos_table, B - 1)
    sc_out = _sc_part(inputs.reshape(B * S * D), pos_table.reshape(S * D))
    out = jnp.concatenate([tc_out, sc_out.reshape(S, D)], axis=0)
    return out.reshape(B, S, D)


# speedup vs baseline: 2.6339x; 2.5187x over previous
import functools
import jax
import jax.numpy as jnp
from jax import lax
from jax.experimental import pallas as pl
from jax.experimental.pallas import tpu as pltpu
from jax.experimental.pallas import tpu_sc as plsc

_B = 4
_S = 8192
_D = 1024

_BS = 2048


def _tc_body(in_ref, pos_ref, out_ref):
    out_ref[...] = in_ref[...] + pos_ref[...]


def _tc_part(flat2d, pos_table, n_batches):
    n_s = _S // _BS
    return pl.pallas_call(
        _tc_body,
        grid=(n_s, n_batches),
        in_specs=[
            pl.BlockSpec((_BS, _D), lambda s, b: (b * n_s + s, 0)),
            pl.BlockSpec((_BS, _D), lambda s, b: (s, 0)),
        ],
        out_specs=pl.BlockSpec((_BS, _D), lambda s, b: (b * n_s + s, 0)),
        out_shape=jax.ShapeDtypeStruct((_B * _S, _D), jnp.float32),
        compiler_params=pltpu.CompilerParams(
            dimension_semantics=("arbitrary", "arbitrary"),
        ),
    )(flat2d, pos_table)


_C = 16
_L = 16
_UNROLL = 8
_RING = 4
_DIST = 2


def _sc_body(in_hbm, pos_hbm, out_hbm, *scratch):
    accs = scratch[0:_RING]
    poss = scratch[_RING:2 * _RING]
    in_sems = scratch[2 * _RING:3 * _RING]
    pos_sems = scratch[3 * _RING:4 * _RING]
    out_sems = scratch[4 * _RING:5 * _RING]

    info = plsc.get_sparse_core_info()
    nc = info.num_cores
    wid = lax.axis_index("s") * nc + lax.axis_index("c")
    rows_per_w = _S // (nc * info.num_subcores)
    G = rows_per_w // _C
    wrow = wid * rows_per_w
    irow = (_B - 1) * _S + wrow

    def in_copy(g, slot):
        return pltpu.make_async_copy(
            in_hbm.at[pl.ds(irow + g * _C, _C)], accs[slot], in_sems[slot])

    def pos_copy(g, slot):
        return pltpu.make_async_copy(
            pos_hbm.at[pl.ds(wrow + g * _C, _C)], poss[slot], pos_sems[slot])

    def out_copy(g, slot):
        return pltpu.make_async_copy(
            accs[slot], out_hbm.at[pl.ds(wrow + g * _C, _C)], out_sems[slot])

    for g in range(_DIST):
        in_copy(g, g).start()
        pos_copy(g, g).start()

    def outer(g4, _):
        for gs in range(_RING):
            g = g4 * _RING + gs
            slot = gs
            in_copy(g, slot).wait()
            pos_copy(g, slot).wait()

            nslot = (slot + _DIST) % _RING

            @pl.when(g + _DIST < G)
            def _():
                @pl.when(g + _DIST - _RING >= 0)
                def _():
                    out_copy(g + _DIST - _RING, nslot).wait()

                in_copy(g + _DIST, nslot).start()
                pos_copy(g + _DIST, nslot).start()

            acc = accs[slot]
            pos = poss[slot]

            def add_row(j, _):
                for ci in range(_D // _L):
                    o = ci * _L
                    acc[j, pl.ds(o, _L)] = (
                        acc[j, pl.ds(o, _L)] + pos[j, pl.ds(o, _L)]
                    )
                return ()

            lax.fori_loop(0, _C, add_row, (), unroll=False)
            out_copy(g, slot).start()
        return ()

    lax.fori_loop(0, G // _RING, outer, (), unroll=False)

    for k in range(_RING):
        g = G - _RING + k
        out_copy(g, g % _RING).wait()


def _sc_part(flat2d, pos_table):
    mesh = plsc.VectorSubcoreMesh(core_axis_name="c", subcore_axis_name="s")
    scratch = (
        [pltpu.VMEM((_C, _D), jnp.float32) for _ in range(2 * _RING)]
        + [pltpu.SemaphoreType.DMA for _ in range(3 * _RING)]
    )
    sc_add = functools.partial(
        pl.kernel,
        mesh=mesh,
        out_type=jax.ShapeDtypeStruct((_S, _D), jnp.float32),
        scratch_types=scratch,
    )(_sc_body)
    return sc_add(flat2d, pos_table)


def kernel(inputs, pos_table):
    inputs = inputs.astype(jnp.float32)
    B, S, D = inputs.shape
    flat = inputs.reshape(B * S, D)

    tc_out = _tc_part(flat, pos_table, B - 1)
    sc_out = _sc_part(flat, pos_table)
    out = lax.dynamic_update_slice(tc_out, sc_out, ((B - 1) * S, 0))
    return out.reshape(B, S, D)


# speedup vs baseline: 4.1146x; 1.5622x over previous
import jax
import jax.numpy as jnp
from jax.experimental import pallas as pl
from jax.experimental.pallas import tpu as pltpu

_BS = 2048


def _add_body(in_ref, pos_ref, out_ref):
    out_ref[...] = in_ref[...] + pos_ref[...]


def kernel(inputs, pos_table):
    inputs = inputs.astype(jnp.float32)
    B, S, D = inputs.shape
    n_s = S // _BS
    flat = inputs.reshape(B * S, D)

    out = pl.pallas_call(
        _add_body,
        grid=(n_s, B),
        in_specs=[
            pl.BlockSpec((_BS, D), lambda s, b: (b * n_s + s, 0)),
            pl.BlockSpec((_BS, D), lambda s, b: (s, 0)),
        ],
        out_specs=pl.BlockSpec((_BS, D), lambda s, b: (b * n_s + s, 0)),
        out_shape=jax.ShapeDtypeStruct((B * S, D), jnp.float32),
        compiler_params=pltpu.CompilerParams(
            dimension_semantics=("arbitrary", "arbitrary"),
        ),
    )(flat, pos_table)
    return out.reshape(B, S, D)
